# baseline (speedup 1.0000x reference)
import jax
import jax.numpy as jnp
from jax.experimental import pallas as pl

COUNT = 6
EPS = 1e-05
NUM_CLASSES = 19
C_PAD = 24


def _body(x_ref, y_ref, mean_ref, std_ref, valid_ref):
    n_i = pl.program_id(1)
    nblk = pl.num_programs(1)
    xb = x_ref[0]
    lab = y_ref[0]
    cls = jax.lax.broadcasted_iota(jnp.int32, (C_PAD, xb.shape[1]), 0)
    ohb = (cls == lab).astype(jnp.bfloat16)
    dn = (((1,), (1,)), ((), ()))
    x_hi = xb.astype(jnp.bfloat16)
    x_lo = (xb - x_hi.astype(jnp.float32)).astype(jnp.bfloat16)
    xsq = (xb * xb).astype(jnp.bfloat16)
    ps = (
        jax.lax.dot_general(ohb, x_hi, dn, preferred_element_type=jnp.float32)
        + jax.lax.dot_general(ohb, x_lo, dn, preferred_element_type=jnp.float32)
    )
    ps2 = jax.lax.dot_general(ohb, xsq, dn, preferred_element_type=jnp.float32)
    pc = jnp.sum(ohb.astype(jnp.float32), axis=1, keepdims=True)

    @pl.when(n_i == 0)
    def _():
        mean_ref[0] = ps
        std_ref[0] = ps2
        valid_ref[0] = pc

    @pl.when(n_i != 0)
    def _():
        mean_ref[0] += ps
        std_ref[0] += ps2
        valid_ref[0] += pc

    @pl.when(n_i == nblk - 1)
    def _():
        s = mean_ref[0]
        s2 = std_ref[0]
        cnt = valid_ref[0]
        safe = jnp.maximum(cnt, 1.0)
        mean = s / safe
        denom = jnp.maximum(cnt - 1.0, 1.0)
        var = jnp.maximum((s2 - safe * mean * mean) / denom, 0.0)
        std = jnp.sqrt(var) + EPS
        v = cnt > float(COUNT)
        mean_ref[0] = jnp.where(v, mean, 0.0)
        std_ref[0] = jnp.where(v, std, 0.0)
        valid_ref[0] = v.astype(jnp.float32)


def kernel(x, y):
    B, d, N = x.shape
    NB = 8192
    nblk = N // NB
    y3 = y.reshape(B, 1, N)
    out_mean, out_std, out_valid = pl.pallas_call(
        _body,
        grid=(B, nblk),
        in_specs=[
            pl.BlockSpec((1, d, NB), lambda b, n: (b, 0, n)),
            pl.BlockSpec((1, 1, NB), lambda b, n: (b, 0, n)),
        ],
        out_specs=[
            pl.BlockSpec((1, C_PAD, d), lambda b, n: (b, 0, 0)),
            pl.BlockSpec((1, C_PAD, d), lambda b, n: (b, 0, 0)),
            pl.BlockSpec((1, C_PAD, 1), lambda b, n: (b, 0, 0)),
        ],
        out_shape=[
            jax.ShapeDtypeStruct((B, C_PAD, d), jnp.float32),
            jax.ShapeDtypeStruct((B, C_PAD, d), jnp.float32),
            jax.ShapeDtypeStruct((B, C_PAD, 1), jnp.float32),
        ],
    )(x, y3)
    means = out_mean[:, :NUM_CLASSES, :]
    stds = out_std[:, :NUM_CLASSES, :]
    valid = out_valid[:, :NUM_CLASSES, 0] > 0.5
    return (means, stds, valid)


# speedup vs baseline: 1.4017x; 1.4017x over previous
import jax
import jax.numpy as jnp
from jax.experimental import pallas as pl

COUNT = 6
EPS = 1e-05
NUM_CLASSES = 19
C_PAD = 24


def _body(x_ref, y_ref, mean_ref, std_ref, valid_ref):
    n_i = pl.program_id(1)
    nblk = pl.num_programs(1)
    xb = x_ref[0]
    lab = y_ref[0]
    cls = jax.lax.broadcasted_iota(jnp.int32, (C_PAD, xb.shape[1]), 0)
    oh = (cls == lab).astype(jnp.float32)
    dn = (((1,), (1,)), ((), ()))
    ps = jax.lax.dot_general(oh, xb, dn, preferred_element_type=jnp.float32)
    ps2 = jax.lax.dot_general(oh, xb * xb, dn, preferred_element_type=jnp.float32)
    pc = jnp.sum(oh, axis=1, keepdims=True)

    @pl.when(n_i == 0)
    def _():
        mean_ref[0] = ps
        std_ref[0] = ps2
        valid_ref[0] = pc

    @pl.when(n_i != 0)
    def _():
        mean_ref[0] += ps
        std_ref[0] += ps2
        valid_ref[0] += pc

    @pl.when(n_i == nblk - 1)
    def _():
        s = mean_ref[0]
        s2 = std_ref[0]
        cnt = valid_ref[0]
        safe = jnp.maximum(cnt, 1.0)
        mean = s / safe
        denom = jnp.maximum(cnt - 1.0, 1.0)
        var = jnp.maximum((s2 - safe * mean * mean) / denom, 0.0)
        std = jnp.sqrt(var) + EPS
        v = cnt > float(COUNT)
        mean_ref[0] = jnp.where(v, mean, 0.0)
        std_ref[0] = jnp.where(v, std, 0.0)
        valid_ref[0] = v.astype(jnp.float32)


def kernel(x, y):
    B, d, N = x.shape
    NB = 16384
    nblk = N // NB
    y3 = y.reshape(B, 1, N)
    out_mean, out_std, out_valid = pl.pallas_call(
        _body,
        grid=(B, nblk),
        in_specs=[
            pl.BlockSpec((1, d, NB), lambda b, n: (b, 0, n)),
            pl.BlockSpec((1, 1, NB), lambda b, n: (b, 0, n)),
        ],
        out_specs=[
            pl.BlockSpec((1, C_PAD, d), lambda b, n: (b, 0, 0)),
            pl.BlockSpec((1, C_PAD, d), lambda b, n: (b, 0, 0)),
            pl.BlockSpec((1, C_PAD, 1), lambda b, n: (b, 0, 0)),
        ],
        out_shape=[
            jax.ShapeDtypeStruct((B, C_PAD, d), jnp.float32),
            jax.ShapeDtypeStruct((B, C_PAD, d), jnp.float32),
            jax.ShapeDtypeStruct((B, C_PAD, 1), jnp.float32),
        ],
    )(x, y3)
    means = out_mean[:, :NUM_CLASSES, :]
    stds = out_std[:, :NUM_CLASSES, :]
    valid = out_valid[:, :NUM_CLASSES, 0] > 0.5
    return (means, stds, valid)
